# baseline (device time: 13321 ns/iter reference)
import jax
import jax.numpy as jnp
from jax import lax
from jax.experimental import pallas as pl
from jax.experimental.pallas import tpu as pltpu

V_PER = 4096
T = 512
D = 512
N_CHUNK = 8
ROWS = T // N_CHUNK


def kernel(ids, E):
    my_x = lax.axis_index("x")
    local = ids - my_x * V_PER
    owned = (local >= 0) & (local < V_PER)
    safe = jnp.where(owned, local, 0)
    gathered = jnp.take(E, safe, axis=0).astype(jnp.bfloat16)
    mask = owned.astype(jnp.int32)[:, None]

    def body(pb_ref, mask_ref, out_ref, acc_ref, recv_ref,
             send_sems, recv_sems, out_sems):
        mx = lax.axis_index("x")
        my = lax.axis_index("y")
        mz = lax.axis_index("z")
        partner = (1 - mx, my, mz)

        barrier = pltpu.get_barrier_semaphore()
        pl.semaphore_signal(
            barrier, inc=1, device_id=partner,
            device_id_type=pl.DeviceIdType.MESH,
        )
        pl.semaphore_wait(barrier, 1)

        rdmas = []
        for c in range(N_CHUNK):
            sl = pl.ds(c * ROWS, ROWS)
            rdma = pltpu.make_async_remote_copy(
                src_ref=pb_ref.at[sl],
                dst_ref=recv_ref.at[sl],
                send_sem=send_sems.at[c],
                recv_sem=recv_sems.at[c],
                device_id=partner,
                device_id_type=pl.DeviceIdType.MESH,
            )
            rdma.start()
            rdmas.append(rdma)

        copies = []
        for c in range(N_CHUNK):
            sl = pl.ds(c * ROWS, ROWS)
            rdmas[c].wait_recv()
            acc_ref[sl, :] = jnp.where(
                mask_ref[sl, :] != 0, pb_ref[sl, :], recv_ref[sl, :]
            ).astype(jnp.float32)
            cp = pltpu.make_async_copy(
                acc_ref.at[sl], out_ref.at[sl], out_sems.at[c]
            )
            cp.start()
            copies.append(cp)

        for c in range(N_CHUNK):
            copies[c].wait()
            rdmas[c].wait_send()

    return pl.pallas_call(
        body,
        out_shape=jax.ShapeDtypeStruct((T, D), jnp.float32),
        in_specs=[
            pl.BlockSpec(memory_space=pltpu.VMEM),
            pl.BlockSpec(memory_space=pltpu.VMEM),
        ],
        out_specs=pl.BlockSpec(memory_space=pl.ANY),
        scratch_shapes=[
            pltpu.VMEM((T, D), jnp.float32),
            pltpu.VMEM((T, D), jnp.bfloat16),
            pltpu.SemaphoreType.DMA((N_CHUNK,)),
            pltpu.SemaphoreType.DMA((N_CHUNK,)),
            pltpu.SemaphoreType.DMA((N_CHUNK,)),
        ],
        compiler_params=pltpu.CompilerParams(collective_id=0),
    )(gathered, mask)


# device time: 13116 ns/iter; 1.0156x vs baseline; 1.0156x over previous
import jax
import jax.numpy as jnp
from jax import lax
from jax.experimental import pallas as pl
from jax.experimental.pallas import tpu as pltpu

V_PER = 4096
T = 512
D = 512
N_CHUNK = 8
ROWS = T // N_CHUNK


def kernel(ids, E):
    my_x = lax.axis_index("x")
    local = ids - my_x * V_PER
    owned = (local >= 0) & (local < V_PER)
    safe = jnp.where(owned, local, 0)
    partial = jnp.where(owned[:, None], jnp.take(E, safe, axis=0), 0.0)
    partial_bf16 = partial.astype(jnp.bfloat16)

    def body(pb_ref, out_ref, acc_ref, recv_ref,
             send_sems, recv_sems, out_sems):
        mx = lax.axis_index("x")
        my = lax.axis_index("y")
        mz = lax.axis_index("z")
        partner = (1 - mx, my, mz)

        barrier = pltpu.get_barrier_semaphore()
        pl.semaphore_signal(
            barrier, inc=1, device_id=partner,
            device_id_type=pl.DeviceIdType.MESH,
        )
        pl.semaphore_wait(barrier, 1)

        rdmas = []
        for c in range(N_CHUNK):
            sl = pl.ds(c * ROWS, ROWS)
            rdma = pltpu.make_async_remote_copy(
                src_ref=pb_ref.at[sl],
                dst_ref=recv_ref.at[sl],
                send_sem=send_sems.at[c],
                recv_sem=recv_sems.at[c],
                device_id=partner,
                device_id_type=pl.DeviceIdType.MESH,
            )
            rdma.start()
            rdmas.append(rdma)

        copies = []
        for c in range(N_CHUNK):
            sl = pl.ds(c * ROWS, ROWS)
            rdmas[c].wait_recv()
            acc_ref[sl, :] = (pb_ref[sl, :] + recv_ref[sl, :]).astype(
                jnp.float32
            )
            cp = pltpu.make_async_copy(
                acc_ref.at[sl], out_ref.at[sl], out_sems.at[c]
            )
            cp.start()
            copies.append(cp)

        for c in range(N_CHUNK):
            copies[c].wait()
            rdmas[c].wait_send()

    return pl.pallas_call(
        body,
        out_shape=jax.ShapeDtypeStruct((T, D), jnp.float32),
        in_specs=[pl.BlockSpec(memory_space=pltpu.VMEM)],
        out_specs=pl.BlockSpec(memory_space=pl.ANY),
        scratch_shapes=[
            pltpu.VMEM((T, D), jnp.float32),
            pltpu.VMEM((T, D), jnp.bfloat16),
            pltpu.SemaphoreType.DMA((N_CHUNK,)),
            pltpu.SemaphoreType.DMA((N_CHUNK,)),
            pltpu.SemaphoreType.DMA((N_CHUNK,)),
        ],
        compiler_params=pltpu.CompilerParams(collective_id=0),
    )(partial_bf16)


# device time: 12773 ns/iter; 1.0429x vs baseline; 1.0269x over previous
import jax
import jax.numpy as jnp
from jax import lax
from jax.experimental import pallas as pl
from jax.experimental.pallas import tpu as pltpu

V_PER = 4096
T = 512
D = 512
N_CHUNK = 4
ROWS = T // N_CHUNK


def kernel(ids, E):
    my_x = lax.axis_index("x")
    local = ids - my_x * V_PER
    partial_bf16 = jnp.take(
        E, local, axis=0, mode="fill", fill_value=0.0
    ).astype(jnp.bfloat16)

    def body(pb_ref, out_ref, recv_ref, send_sems, recv_sems):
        mx = lax.axis_index("x")
        my = lax.axis_index("y")
        mz = lax.axis_index("z")
        partner = (1 - mx, my, mz)

        barrier = pltpu.get_barrier_semaphore()
        pl.semaphore_signal(
            barrier, inc=1, device_id=partner,
            device_id_type=pl.DeviceIdType.MESH,
        )
        pl.semaphore_wait(barrier, 1)

        rdmas = []
        for c in range(N_CHUNK):
            sl = pl.ds(c * ROWS, ROWS)
            rdma = pltpu.make_async_remote_copy(
                src_ref=pb_ref.at[sl],
                dst_ref=recv_ref.at[sl],
                send_sem=send_sems.at[c],
                recv_sem=recv_sems.at[c],
                device_id=partner,
                device_id_type=pl.DeviceIdType.MESH,
            )
            rdma.start()
            rdmas.append(rdma)

        for c in range(N_CHUNK):
            sl = pl.ds(c * ROWS, ROWS)
            rdmas[c].wait_recv()
            out_ref[sl, :] = (pb_ref[sl, :] + recv_ref[sl, :]).astype(
                jnp.float32
            )
        for c in range(N_CHUNK):
            rdmas[c].wait_send()

    return pl.pallas_call(
        body,
        out_shape=jax.ShapeDtypeStruct((T, D), jnp.float32),
        in_specs=[pl.BlockSpec(memory_space=pltpu.VMEM)],
        out_specs=pl.BlockSpec(memory_space=pltpu.VMEM),
        scratch_shapes=[
            pltpu.VMEM((T, D), jnp.bfloat16),
            pltpu.SemaphoreType.DMA((N_CHUNK,)),
            pltpu.SemaphoreType.DMA((N_CHUNK,)),
        ],
        compiler_params=pltpu.CompilerParams(collective_id=0),
    )(partial_bf16)
